# baseline (speedup 1.0000x reference)
import functools

import jax
import jax.numpy as jnp
from jax import lax
from jax.experimental import pallas as pl
from jax.experimental.pallas import tpu as pltpu
from jax.experimental.pallas import tpu_sc as plsc

HIDDEN = 2048
NUM_EXPERTS = 16
TOP_K = 2
BETA = 0.1
GAMMA = 0.1
EXPLORATION_C = 0.1
LOAD_EMA_ALPHA = 0.9

TB = 1024
T_TOTAL = 8192
NW = 32
TPW = T_TOTAL // NW
NG = TPW // 16
E = NUM_EXPERTS
L = 16


def _logits_block(x_ref, w_ref, bias_ref, out_ref):
    out_ref[...] = jax.lax.dot_general(
        x_ref[...], w_ref[...],
        dimension_numbers=(((1,), (1,)), ((), ())),
        preferred_element_type=jnp.float32) + bias_ref[...]


def _tc_logits(x2, W, bias):
    T = x2.shape[0]
    return pl.pallas_call(
        _logits_block,
        grid=(T // TB,),
        in_specs=[
            pl.BlockSpec((TB, HIDDEN), lambda i: (i, 0)),
            pl.BlockSpec((E, HIDDEN), lambda i: (0, 0)),
            pl.BlockSpec((1, E), lambda i: (0, 0)),
        ],
        out_specs=pl.BlockSpec((TB, E), lambda i: (i, 0)),
        out_shape=jax.ShapeDtypeStruct((T, E), jnp.float32),
    )(x2, W, bias)


_sc_mesh = plsc.VectorSubcoreMesh(core_axis_name="c", subcore_axis_name="s")


@functools.partial(
    pl.kernel,
    mesh=_sc_mesh,
    compiler_params=pltpu.CompilerParams(needs_layout_passes=False),
    out_type=[
        jax.ShapeDtypeStruct((2 * T_TOTAL,), jnp.float32),
        jax.ShapeDtypeStruct((2 * T_TOTAL,), jnp.int32),
        jax.ShapeDtypeStruct((NW * 2 * L,), jnp.float32),
    ],
    scratch_types=[
        pltpu.VMEM((TPW * E,), jnp.float32),
        pltpu.VMEM((2 * TPW,), jnp.float32),
        pltpu.VMEM((2 * TPW,), jnp.int32),
        pltpu.VMEM((E,), jnp.float32),
        pltpu.VMEM((E * L,), jnp.float32),
        pltpu.VMEM((2 * L,), jnp.float32),
    ],
)
def _sc_route(lg_hbm, wout_hbm, iout_hbm, part_hbm,
              lg_v, w_v, i_v, cnt_v, pm_v, part_v):
    wid = lax.axis_index("s") * 2 + lax.axis_index("c")
    base = wid * TPW
    pltpu.sync_copy(lg_hbm.at[pl.ds(base * E, TPW * E)], lg_v)

    lane = lax.broadcasted_iota(jnp.int32, (L,), 0)
    ones = jnp.ones((L,), jnp.float32)
    cnt_v[...] = jnp.zeros((E,), jnp.float32)
    probacc = [jnp.zeros((L,), jnp.float32) for _ in range(E)]

    for g in range(NG):
        tok = lane + (g * L)
        tok_e = tok * E
        ls = [plsc.load_gather(lg_v, [tok_e + e]) for e in range(E)]
        m1 = ls[0]
        i1 = jnp.zeros((L,), jnp.int32)
        m2 = jnp.full((L,), -jnp.inf, jnp.float32)
        i2 = jnp.zeros((L,), jnp.int32)
        for e in range(1, E):
            l = ls[e]
            e_v = jnp.full((L,), e, jnp.int32)
            gt1 = l > m1
            gt2 = l > m2
            m2n = jnp.where(gt2, l, m2)
            i2n = jnp.where(gt2, e_v, i2)
            m2 = jnp.where(gt1, m1, m2n)
            i2 = jnp.where(gt1, i1, i2n)
            m1 = jnp.where(gt1, l, m1)
            i1 = jnp.where(gt1, e_v, i1)
        exps = [jnp.exp(l - m1) for l in ls]
        denom = exps[0]
        for e in range(1, E):
            denom = denom + exps[e]
        rden = 1.0 / denom
        for e in range(E):
            probacc[e] = probacc[e] + exps[e] * rden
        e2 = jnp.exp(m2 - m1)
        w1 = 1.0 / (1.0 + e2)
        w2 = 1.0 - w1
        pos = tok * 2
        plsc.store_scatter(w_v, [pos], w1)
        plsc.store_scatter(w_v, [pos + 1], w2)
        plsc.store_scatter(i_v, [pos], i1)
        plsc.store_scatter(i_v, [pos + 1], i2)
        plsc.addupdate_scatter(cnt_v, [i1], ones)
        plsc.addupdate_scatter(cnt_v, [i2], ones)

    for e in range(E):
        pm_v[pl.ds(e * L, L)] = probacc[e]
    lane_l = lane * L
    psum = jnp.zeros((L,), jnp.float32)
    for t in range(L):
        psum = psum + plsc.load_gather(pm_v, [lane_l + t])

    part_v[pl.ds(0, L)] = cnt_v[...]
    part_v[pl.ds(L, L)] = psum
    pltpu.sync_copy(w_v, wout_hbm.at[pl.ds(base * 2, 2 * TPW)])
    pltpu.sync_copy(i_v, iout_hbm.at[pl.ds(base * 2, 2 * TPW)])
    pltpu.sync_copy(part_v, part_hbm.at[pl.ds(wid * 2 * L, 2 * L)])


def kernel(x, W, reputation_scores, expert_loads, expert_counts,
           total_routing_decisions):
    B, S, H = x.shape
    x2 = x.reshape(-1, H)
    updated_loads = (LOAD_EMA_ALPHA * expert_loads
                     + (1.0 - LOAD_EMA_ALPHA) * expert_loads)
    exploration = EXPLORATION_C * jnp.sqrt(
        jnp.log(total_routing_decisions + 1.0) / (expert_counts + 1e-10))
    bias = (BETA * reputation_scores - GAMMA * updated_loads
            + exploration).reshape(1, E).astype(jnp.float32)

    lg = _tc_logits(x2, W, bias)
    wflat, iflat, part = _sc_route(lg.reshape(-1))

    routing_weights = wflat.reshape(B, S, TOP_K)
    expert_indices = iflat.reshape(B, S, TOP_K)
    part = part.reshape(NW, 2, E)
    cnt_tot = jnp.sum(part[:, 0, :], axis=0)
    prob_tot = jnp.sum(part[:, 1, :], axis=0)
    tf = jnp.float32(B * S)
    aux = jnp.sum(cnt_tot * prob_tot) * NUM_EXPERTS / (tf * tf)
    return routing_weights, expert_indices, aux


# speedup vs baseline: 1.0236x; 1.0236x over previous
import functools

import jax
import jax.numpy as jnp
from jax import lax
from jax.experimental import pallas as pl
from jax.experimental.pallas import tpu as pltpu
from jax.experimental.pallas import tpu_sc as plsc

HIDDEN = 2048
NUM_EXPERTS = 16
TOP_K = 2
BETA = 0.1
GAMMA = 0.1
EXPLORATION_C = 0.1
LOAD_EMA_ALPHA = 0.9

TB = 1024
T_TOTAL = 8192
NW = 32
TPW = T_TOTAL // NW
NG = TPW // 16
E = NUM_EXPERTS
L = 16


def _logits_block(x_ref, w_ref, bias_ref, out_ref):
    out_ref[...] = jax.lax.dot_general(
        x_ref[...], w_ref[...],
        dimension_numbers=(((1,), (1,)), ((), ())),
        preferred_element_type=jnp.float32) + bias_ref[...]


def _tc_logits(x2, W, bias):
    T = x2.shape[0]
    return pl.pallas_call(
        _logits_block,
        grid=(T // TB,),
        in_specs=[
            pl.BlockSpec((TB, HIDDEN), lambda i: (i, 0)),
            pl.BlockSpec((E, HIDDEN), lambda i: (0, 0)),
            pl.BlockSpec((1, E), lambda i: (0, 0)),
        ],
        out_specs=pl.BlockSpec((TB, E), lambda i: (i, 0)),
        out_shape=jax.ShapeDtypeStruct((T, E), jnp.float32),
    )(x2, W, bias)


_sc_mesh = plsc.VectorSubcoreMesh(core_axis_name="c", subcore_axis_name="s")


@functools.partial(
    pl.kernel,
    mesh=_sc_mesh,
    compiler_params=pltpu.CompilerParams(needs_layout_passes=False),
    out_type=[
        jax.ShapeDtypeStruct((2 * T_TOTAL,), jnp.float32),
        jax.ShapeDtypeStruct((2 * T_TOTAL,), jnp.int32),
        jax.ShapeDtypeStruct((NW * 2 * L,), jnp.float32),
    ],
    scratch_types=[
        pltpu.VMEM((TPW * E,), jnp.float32),
        pltpu.VMEM((2 * TPW,), jnp.float32),
        pltpu.VMEM((2 * TPW,), jnp.int32),
        pltpu.VMEM((E,), jnp.float32),
        pltpu.VMEM((E * L,), jnp.float32),
        pltpu.VMEM((2 * L,), jnp.float32),
    ],
)
def _sc_route(lg_hbm, wout_hbm, iout_hbm, part_hbm,
              lg_v, w_v, i_v, cnt_v, pm_v, part_v):
    wid = lax.axis_index("s") * 2 + lax.axis_index("c")
    base = wid * TPW
    pltpu.sync_copy(lg_hbm.at[pl.ds(base * E, TPW * E)], lg_v)

    lane = lax.broadcasted_iota(jnp.int32, (L,), 0)
    ones = jnp.ones((L,), jnp.float32)
    zeros = jnp.zeros((L,), jnp.float32)
    cnt_v[...] = jnp.zeros((E,), jnp.float32)
    for e in range(E):
        pm_v[pl.ds(e * L, L)] = zeros

    @pl.loop(0, NG)
    def _group(g):
        tok = lane + g * L
        tok_e = tok * E
        ls = [plsc.load_gather(lg_v, [tok_e + e]) for e in range(E)]
        m1 = ls[0]
        i1 = jnp.zeros((L,), jnp.int32)
        m2 = jnp.full((L,), -jnp.inf, jnp.float32)
        i2 = jnp.zeros((L,), jnp.int32)
        for e in range(1, E):
            l = ls[e]
            e_v = jnp.full((L,), e, jnp.int32)
            gt1 = l > m1
            gt2 = l > m2
            m2n = jnp.where(gt2, l, m2)
            i2n = jnp.where(gt2, e_v, i2)
            m2 = jnp.where(gt1, m1, m2n)
            i2 = jnp.where(gt1, i1, i2n)
            m1 = jnp.where(gt1, l, m1)
            i1 = jnp.where(gt1, e_v, i1)
        exps = [jnp.exp(l - m1) for l in ls]
        denom = exps[0]
        for e in range(1, E):
            denom = denom + exps[e]
        rden = 1.0 / denom
        for e in range(E):
            plsc.addupdate(pm_v.at[pl.ds(e * L, L)], exps[e] * rden)
        e2 = jnp.exp(m2 - m1)
        w1 = 1.0 / (1.0 + e2)
        w2 = 1.0 - w1
        pos = tok * 2
        plsc.store_scatter(w_v, [pos], w1)
        plsc.store_scatter(w_v, [pos + 1], w2)
        plsc.store_scatter(i_v, [pos], i1)
        plsc.store_scatter(i_v, [pos + 1], i2)
        plsc.addupdate_scatter(cnt_v, [i1], ones)
        plsc.addupdate_scatter(cnt_v, [i2], ones)

    lane_l = lane * L
    psum = jnp.zeros((L,), jnp.float32)
    for t in range(L):
        psum = psum + plsc.load_gather(pm_v, [lane_l + t])

    part_v[pl.ds(0, L)] = cnt_v[...]
    part_v[pl.ds(L, L)] = psum
    pltpu.sync_copy(w_v, wout_hbm.at[pl.ds(base * 2, 2 * TPW)])
    pltpu.sync_copy(i_v, iout_hbm.at[pl.ds(base * 2, 2 * TPW)])
    pltpu.sync_copy(part_v, part_hbm.at[pl.ds(wid * 2 * L, 2 * L)])


def kernel(x, W, reputation_scores, expert_loads, expert_counts,
           total_routing_decisions):
    B, S, H = x.shape
    x2 = x.reshape(-1, H)
    updated_loads = (LOAD_EMA_ALPHA * expert_loads
                     + (1.0 - LOAD_EMA_ALPHA) * expert_loads)
    exploration = EXPLORATION_C * jnp.sqrt(
        jnp.log(total_routing_decisions + 1.0) / (expert_counts + 1e-10))
    bias = (BETA * reputation_scores - GAMMA * updated_loads
            + exploration).reshape(1, E).astype(jnp.float32)

    lg = _tc_logits(x2, W, bias)
    wflat, iflat, part = _sc_route(lg.reshape(-1))

    routing_weights = wflat.reshape(B, S, TOP_K)
    expert_indices = iflat.reshape(B, S, TOP_K)
    part = part.reshape(NW, 2, E)
    cnt_tot = jnp.sum(part[:, 0, :], axis=0)
    prob_tot = jnp.sum(part[:, 1, :], axis=0)
    tf = jnp.float32(B * S)
    aux = jnp.sum(cnt_tot * prob_tot) * NUM_EXPERTS / (tf * tf)
    return routing_weights, expert_indices, aux


# speedup vs baseline: 1.1492x; 1.1227x over previous
import functools

import jax
import jax.numpy as jnp
from jax import lax
from jax.experimental import pallas as pl
from jax.experimental.pallas import tpu as pltpu
from jax.experimental.pallas import tpu_sc as plsc

HIDDEN = 2048
NUM_EXPERTS = 16
TOP_K = 2
BETA = 0.1
GAMMA = 0.1
EXPLORATION_C = 0.1
LOAD_EMA_ALPHA = 0.9

TB = 1024
T_TOTAL = 8192
NW = 32
TPW = T_TOTAL // NW
NG = TPW // 16
E = NUM_EXPERTS
L = 16


def _logits_block(x_ref, w_ref, bias_ref, out_ref):
    out_ref[...] = jax.lax.dot_general(
        x_ref[...], w_ref[...],
        dimension_numbers=(((1,), (1,)), ((), ())),
        preferred_element_type=jnp.float32) + bias_ref[...]


def _tc_logits(x2, W, bias):
    T = x2.shape[0]
    return pl.pallas_call(
        _logits_block,
        grid=(T // TB,),
        in_specs=[
            pl.BlockSpec((TB, HIDDEN), lambda i: (i, 0)),
            pl.BlockSpec((E, HIDDEN), lambda i: (0, 0)),
            pl.BlockSpec((1, E), lambda i: (0, 0)),
        ],
        out_specs=pl.BlockSpec((TB, E), lambda i: (i, 0)),
        out_shape=jax.ShapeDtypeStruct((T, E), jnp.float32),
    )(x2, W, bias)


_sc_mesh = plsc.VectorSubcoreMesh(core_axis_name="c", subcore_axis_name="s")


@functools.partial(
    pl.kernel,
    mesh=_sc_mesh,
    compiler_params=pltpu.CompilerParams(needs_layout_passes=False),
    out_type=[
        jax.ShapeDtypeStruct((T_TOTAL, TOP_K), jnp.float32),
        jax.ShapeDtypeStruct((T_TOTAL, TOP_K), jnp.int32),
        jax.ShapeDtypeStruct((NW * 2 * L,), jnp.float32),
    ],
    scratch_types=[
        pltpu.VMEM((TPW, E), jnp.float32),
        pltpu.VMEM((TPW, TOP_K), jnp.float32),
        pltpu.VMEM((TPW, TOP_K), jnp.int32),
        pltpu.VMEM((E,), jnp.float32),
        pltpu.VMEM((E * L,), jnp.float32),
        pltpu.VMEM((2 * L,), jnp.float32),
    ],
)
def _sc_route(lg_hbm, wout_hbm, iout_hbm, part_hbm,
              lg_v, w_v, i_v, cnt_v, pm_v, part_v):
    wid = lax.axis_index("s") * 2 + lax.axis_index("c")
    base = wid * TPW
    pltpu.sync_copy(lg_hbm.at[pl.ds(base, TPW)], lg_v)

    lane = lax.broadcasted_iota(jnp.int32, (L,), 0)
    ones = jnp.ones((L,), jnp.float32)
    zeros = jnp.zeros((L,), jnp.float32)
    cnt_v[...] = jnp.zeros((E,), jnp.float32)
    for e in range(E):
        pm_v[pl.ds(e * L, L)] = zeros

    zeros_i = jnp.zeros((L,), jnp.int32)
    ones_i = jnp.full((L,), 1, jnp.int32)
    e_vecs = [jnp.full((L,), e, jnp.int32) for e in range(E)]

    @pl.loop(0, NG)
    def _group(g):
        tok = lane + g * L
        ls = [plsc.load_gather(lg_v, [tok, e_vecs[e]]) for e in range(E)]
        m1 = ls[0]
        i1 = jnp.zeros((L,), jnp.int32)
        m2 = jnp.full((L,), -jnp.inf, jnp.float32)
        i2 = jnp.zeros((L,), jnp.int32)
        for e in range(1, E):
            l = ls[e]
            e_v = e_vecs[e]
            gt1 = l > m1
            gt2 = l > m2
            m2n = jnp.where(gt2, l, m2)
            i2n = jnp.where(gt2, e_v, i2)
            m2 = jnp.where(gt1, m1, m2n)
            i2 = jnp.where(gt1, i1, i2n)
            m1 = jnp.where(gt1, l, m1)
            i1 = jnp.where(gt1, e_v, i1)
        exps = [jnp.exp(l - m1) for l in ls]
        denom = exps[0]
        for e in range(1, E):
            denom = denom + exps[e]
        rden = 1.0 / denom
        for e in range(E):
            plsc.addupdate(pm_v.at[pl.ds(e * L, L)], exps[e] * rden)
        e2 = jnp.exp(m2 - m1)
        w1 = 1.0 / (1.0 + e2)
        w2 = 1.0 - w1
        plsc.store_scatter(w_v, [tok, zeros_i], w1)
        plsc.store_scatter(w_v, [tok, ones_i], w2)
        plsc.store_scatter(i_v, [tok, zeros_i], i1)
        plsc.store_scatter(i_v, [tok, ones_i], i2)
        plsc.addupdate_scatter(cnt_v, [i1], ones)
        plsc.addupdate_scatter(cnt_v, [i2], ones)

    lane_l = lane * L
    psum = jnp.zeros((L,), jnp.float32)
    for t in range(L):
        psum = psum + plsc.load_gather(pm_v, [lane_l + t])

    part_v[pl.ds(0, L)] = cnt_v[...]
    part_v[pl.ds(L, L)] = psum
    pltpu.sync_copy(w_v, wout_hbm.at[pl.ds(base, TPW)])
    pltpu.sync_copy(i_v, iout_hbm.at[pl.ds(base, TPW)])
    pltpu.sync_copy(part_v, part_hbm.at[pl.ds(wid * 2 * L, 2 * L)])


def kernel(x, W, reputation_scores, expert_loads, expert_counts,
           total_routing_decisions):
    B, S, H = x.shape
    x2 = x.reshape(-1, H)
    updated_loads = (LOAD_EMA_ALPHA * expert_loads
                     + (1.0 - LOAD_EMA_ALPHA) * expert_loads)
    exploration = EXPLORATION_C * jnp.sqrt(
        jnp.log(total_routing_decisions + 1.0) / (expert_counts + 1e-10))
    bias = (BETA * reputation_scores - GAMMA * updated_loads
            + exploration).reshape(1, E).astype(jnp.float32)

    lg = _tc_logits(x2, W, bias)
    wflat, iflat, part = _sc_route(lg)

    routing_weights = wflat.reshape(B, S, TOP_K)
    expert_indices = iflat.reshape(B, S, TOP_K)
    part = part.reshape(NW, 2, E)
    cnt_tot = jnp.sum(part[:, 0, :], axis=0)
    prob_tot = jnp.sum(part[:, 1, :], axis=0)
    tf = jnp.float32(B * S)
    aux = jnp.sum(cnt_tot * prob_tot) * NUM_EXPERTS / (tf * tf)
    return routing_weights, expert_indices, aux


# speedup vs baseline: 1.9901x; 1.7317x over previous
import functools

import jax
import jax.numpy as jnp
from jax.experimental import pallas as pl
from jax.experimental.pallas import tpu as pltpu

HIDDEN = 2048
NUM_EXPERTS = 16
TOP_K = 2
BETA = 0.1
GAMMA = 0.1
EXPLORATION_C = 0.1
LOAD_EMA_ALPHA = 0.9

TB = 1024
E = NUM_EXPERTS


def _router_block(x_ref, w_ref, bias_ref, wout_ref, iout_ref, aux_ref,
                  acc_ref):
    step = pl.program_id(0)
    nsteps = pl.num_programs(0)

    @pl.when(step == 0)
    def _():
        acc_ref[...] = jnp.zeros_like(acc_ref)

    lgT = jax.lax.dot_general(
        w_ref[...], x_ref[...],
        dimension_numbers=(((1,), (1,)), ((), ())),
        preferred_element_type=jnp.float32) + bias_ref[:, 0:1]

    iota_e = jax.lax.broadcasted_iota(jnp.int32, (E, TB), 0)

    m1 = jnp.max(lgT, axis=0, keepdims=True)
    i1 = jnp.min(jnp.where(lgT == m1, iota_e, E), axis=0,
                 keepdims=True)
    masked = jnp.where(iota_e == i1, -jnp.inf, lgT)
    m2 = jnp.max(masked, axis=0, keepdims=True)
    i2 = jnp.min(jnp.where(masked == m2, iota_e, E), axis=0, keepdims=True)

    e2 = jnp.exp(m2 - m1)
    w1 = 1.0 / (1.0 + e2)
    w2 = 1.0 - w1

    wout_ref[...] = jnp.concatenate([w1, w2], axis=0).T
    iout_ref[...] = jnp.concatenate([i1, i2], axis=0).T

    p = jnp.exp(lgT - m1)
    probs = p / jnp.sum(p, axis=0, keepdims=True)
    prob_sum = jnp.sum(probs, axis=1, keepdims=True)
    gate = ((iota_e == i1).astype(jnp.float32)
            + (iota_e == i2).astype(jnp.float32))
    cnt_sum = jnp.sum(gate, axis=1, keepdims=True)

    acc_ref[:, 0:1] += prob_sum
    acc_ref[:, 1:2] += cnt_sum

    @pl.when(step == nsteps - 1)
    def _():
        total_t = jnp.float32(TB) * nsteps
        aux = (jnp.sum(acc_ref[:, 0:1] * acc_ref[:, 1:2])
               * NUM_EXPERTS / (total_t * total_t))
        aux_ref[0, 0] = aux


@jax.jit
def _router(x2, W, bias):
    T = x2.shape[0]
    grid = (T // TB,)
    wout, iout, aux = pl.pallas_call(
        _router_block,
        grid=grid,
        in_specs=[
            pl.BlockSpec((TB, HIDDEN), lambda i: (i, 0)),
            pl.BlockSpec((E, HIDDEN), lambda i: (0, 0)),
            pl.BlockSpec((E, 128), lambda i: (0, 0)),
        ],
        out_specs=[
            pl.BlockSpec((TB, TOP_K), lambda i: (i, 0)),
            pl.BlockSpec((TB, TOP_K), lambda i: (i, 0)),
            pl.BlockSpec(memory_space=pltpu.SMEM),
        ],
        out_shape=[
            jax.ShapeDtypeStruct((T, TOP_K), jnp.float32),
            jax.ShapeDtypeStruct((T, TOP_K), jnp.int32),
            jax.ShapeDtypeStruct((1, 1), jnp.float32),
        ],
        scratch_shapes=[pltpu.VMEM((E, 128), jnp.float32)],
    )(x2, W, bias)
    return wout, iout, aux


def kernel(x, W, reputation_scores, expert_loads, expert_counts,
           total_routing_decisions):
    B, S, H = x.shape
    x2 = x.reshape(-1, H)
    updated_loads = (LOAD_EMA_ALPHA * expert_loads
                     + (1.0 - LOAD_EMA_ALPHA) * expert_loads)
    exploration = EXPLORATION_C * jnp.sqrt(
        jnp.log(total_routing_decisions + 1.0) / (expert_counts + 1e-10))
    bias = (BETA * reputation_scores - GAMMA * updated_loads
            + exploration).astype(jnp.float32)
    bias2 = jnp.broadcast_to(bias.reshape(E, 1), (E, 128))

    wout, iout, aux = _router(x2, W, bias2)
    routing_weights = wout.reshape(B, S, TOP_K)
    expert_indices = iout.reshape(B, S, TOP_K)
    return routing_weights, expert_indices, aux.reshape(())
